# speedup vs baseline: 1.0110x; 1.0110x over previous
import functools
import jax
import jax.numpy as jnp
from jax.experimental import pallas as pl

N = 10000
E = 160000
D = 256
R = 50
NUM_LAYER = 4
NODE_RATIO = 0.1
NEG = 33

BN = 200


def _dense_tail_body(sum_ref, max_ref, min_ref, sq_ref, deg_ref, hid_ref,
                     wout_ref, bout_ref, w1_ref, b1_ref, w2_ref, b2_ref,
                     pna_ref, hid_out_ref, score_out_ref, li_out_ref):
    deg = deg_ref[...]
    deg_safe = jnp.maximum(deg, 1.0)
    has_edge = deg > 0.0

    s = sum_ref[...]
    mean = s / deg_safe
    sq = sq_ref[...] / deg_safe
    std = jnp.sqrt(jnp.clip(sq - mean * mean, 0.0, None) + 1e-6)
    mx = jnp.where(has_edge, max_ref[...], 0.0)
    mn = jnp.where(has_edge, min_ref[...], 0.0)

    aggs = jnp.concatenate([mean, mx, mn, std], axis=1)

    w_id = wout_ref[0:4 * D, :]
    w_amp = wout_ref[4 * D:8 * D, :]
    w_att = wout_ref[8 * D:12 * D, :]
    acc0 = jnp.dot(aggs, w_id, preferred_element_type=jnp.float32)
    acc1 = jnp.dot(aggs, w_amp, preferred_element_type=jnp.float32)
    acc2 = jnp.dot(aggs, w_att, preferred_element_type=jnp.float32)

    pna_mean = pna_ref[0, 0]
    deg_l = jnp.log(deg + 1.0)
    amp = deg_l / pna_mean
    att = jnp.where(has_edge, pna_mean / jnp.maximum(deg_l, 1e-6), 0.0)

    hidden_out = acc0 + amp * acc1 + att * acc2 + bout_ref[...]
    hidden_new = hid_ref[...] + jnp.where(has_edge, hidden_out, 0.0)
    hid_out_ref[...] = hidden_new

    t = jnp.maximum(
        jnp.dot(hidden_new, w1_ref[...], preferred_element_type=jnp.float32)
        + b1_ref[...], 0.0)
    score = jnp.dot(t, w2_ref[...], preferred_element_type=jnp.float32) + b2_ref[...]
    score_out_ref[...] = score
    li_out_ref[...] = hidden_new / (1.0 + jnp.exp(-score))


@jax.jit
def _dense_tail(agg_sum, agg_max, agg_min, agg_sq, deg, hidden,
                w_out_i, b_out_i, w1, b1, w2, b2, pna_mean):
    grid = (N // BN,)
    row = pl.BlockSpec((BN, D), lambda i: (i, 0))
    col1 = pl.BlockSpec((BN, 1), lambda i: (i, 0))
    full = lambda shape: pl.BlockSpec(shape, lambda i: (0, 0))
    return pl.pallas_call(
        _dense_tail_body,
        grid=grid,
        in_specs=[row, row, row, row, col1, row,
                  full((12 * D, D)), full((1, D)),
                  full((D, 2 * D)), full((1, 2 * D)),
                  full((2 * D, 1)), full((1, 1)),
                  full((1, 1))],
        out_specs=[row, col1, row],
        out_shape=[jax.ShapeDtypeStruct((N, D), jnp.float32),
                   jax.ShapeDtypeStruct((N, 1), jnp.float32),
                   jax.ShapeDtypeStruct((N, D), jnp.float32)],
    )(agg_sum, agg_max, agg_min, agg_sq, deg, hidden,
      w_out_i, b_out_i, w1, b1, w2, b2, pna_mean)


def kernel(h_index, r_index, t_index, hidden_states, rel_hidden_states,
           edge_index, edge_attr, score_text_embs, all_index,
           rel_embedding, msg_rel, W_out, b_out, W1, b1, W2, b2):
    src = jnp.concatenate([edge_index[0], edge_index[1]], axis=0)
    dst = jnp.concatenate([edge_index[1], edge_index[0]], axis=0)
    ea = jnp.concatenate([edge_attr, edge_attr + R], axis=0)
    E2 = 2 * E

    r0 = r_index[:, 0]
    rel_embeds = rel_embedding[r0] + rel_hidden_states[r0]

    boundary = jnp.zeros((N, D), dtype=jnp.float32)
    boundary = boundary.at[h_index[:, 0]].add(rel_embeds + hidden_states)
    boundary = boundary.at[all_index].add(score_text_embs)
    init_score = jnp.zeros((N,), dtype=jnp.float32).at[h_index[:, 0]].set(5.0)

    degree_out = jax.ops.segment_sum(jnp.ones((E2,), jnp.float32), src,
                                     num_segments=N)
    pna_mean = jnp.log(degree_out + 1.0).mean().reshape(1, 1)

    hidden = boundary
    score = init_score
    k_sel = int(NODE_RATIO * E2)
    layer_input = jax.nn.sigmoid(score)[:, None] * hidden

    for i in range(NUM_LAYER):
        edge_scores = score[src]
        _, top_idx = jax.lax.top_k(edge_scores, k_sel)
        s_src = src[top_idx]
        s_dst = dst[top_idx]
        s_ea = ea[top_idx]

        msg = layer_input[s_src] * msg_rel[i][s_ea]

        deg = jax.ops.segment_sum(jnp.ones((k_sel,), jnp.float32), s_dst,
                                  num_segments=N)
        agg_sum = jax.ops.segment_sum(msg, s_dst, num_segments=N)
        agg_max = jax.ops.segment_max(msg, s_dst, num_segments=N)
        agg_min = -jax.ops.segment_max(-msg, s_dst, num_segments=N)
        agg_sq = jax.ops.segment_sum(msg * msg, s_dst, num_segments=N)

        hidden, score2d, layer_input = _dense_tail(
            agg_sum, agg_max, agg_min, agg_sq, deg.reshape(N, 1), hidden,
            W_out[i], b_out[i].reshape(1, D), W1, b1.reshape(1, 2 * D),
            W2, b2.reshape(1, 1), pna_mean)
        score = score2d[:, 0]

    return score[t_index]
